# baseline (speedup 1.0000x reference)
import functools

import jax
import jax.numpy as jnp
from jax import lax
from jax.experimental import pallas as pl
from jax.experimental.pallas import tpu as pltpu
from jax.experimental.pallas import tpu_sc as plsc

_TOP_K = 1
_CAP_FACTOR = 1.25
_BLK = 256


def _router_body(noisy_ref, sidx_ref, gidx_ref, run_ref, *, cap, slots, E, trash):
    i = pl.program_id(0)

    @pl.when(i == 0)
    def _():
        run_ref[...] = jnp.zeros_like(run_ref)

    nb = noisy_ref[...]
    blk = nb.shape[0]
    m = jnp.max(nb, axis=1, keepdims=True)
    lane = lax.broadcasted_iota(jnp.int32, (blk, E), 1)
    eidx = jnp.min(jnp.where(nb == m, lane, E), axis=1, keepdims=True)
    onehot = (lane == eidx).astype(jnp.float32)
    r = lax.broadcasted_iota(jnp.int32, (blk, blk), 0)
    c = lax.broadcasted_iota(jnp.int32, (blk, blk), 1)
    ltri = (r >= c).astype(jnp.float32)
    incl = jnp.dot(ltri, onehot, preferred_element_type=jnp.float32)
    run = run_ref[0:1, :]
    pos = jnp.sum((incl + run - 1.0) * onehot, axis=1, keepdims=True).astype(jnp.int32)
    run_ref[0:1, :] = run + jnp.sum(onehot, axis=0, keepdims=True)
    base = eidx * slots
    sidx_ref[...] = jnp.where(pos < cap, base + pos, trash)
    gidx_ref[...] = base + jnp.minimum(pos, cap)


def _ffn_body(x_ref, w1_ref, w2_ref, out_ref, *, cap):
    h = pl.program_id(1)
    xb = x_ref[...]
    row = lax.broadcasted_iota(jnp.int32, (xb.shape[0], 1), 0)
    xb = jnp.where(row < cap, xb, 0.0)
    a = jnp.dot(xb, w1_ref[0], preferred_element_type=jnp.float32)
    s = a * jax.nn.sigmoid(a)
    contrib = jnp.dot(s, w2_ref[0], preferred_element_type=jnp.float32)

    @pl.when(h == 0)
    def _():
        out_ref[...] = jnp.zeros_like(out_ref)

    out_ref[...] += contrib


def kernel(x, Wr, Wn, W1, W2, noise):
    Bv, Tv, C = x.shape
    E = Wr.shape[1]
    H = W1.shape[2]
    N = Bv * Tv
    cap = int(N * _TOP_K / E * _CAP_FACTOR)
    slots = cap + 8
    trash = E * slots
    nrows = trash + 8

    logits = x @ Wr
    noise_logits = x @ Wn
    noisy = (logits + noise * jax.nn.softplus(noise_logits)).reshape(N, E)

    sidx, gidx = pl.pallas_call(
        functools.partial(_router_body, cap=cap, slots=slots, E=E, trash=trash),
        grid=(N // _BLK,),
        in_specs=[pl.BlockSpec((_BLK, E), lambda i: (i, 0))],
        out_specs=[
            pl.BlockSpec((_BLK, 1), lambda i: (i, 0)),
            pl.BlockSpec((_BLK, 1), lambda i: (i, 0)),
        ],
        out_shape=[jax.ShapeDtypeStruct((N, 1), jnp.int32)] * 2,
        scratch_shapes=[pltpu.VMEM((8, E), jnp.float32)],
    )(noisy)
    sidx = sidx.reshape(N)
    gidx = gidx.reshape(N)

    xf = x.reshape(N, C)
    info = plsc.get_sparse_core_info()
    NW = info.num_cores * info.num_subcores
    tpw = N // NW
    mesh = plsc.VectorSubcoreMesh(core_axis_name="c", subcore_axis_name="s")

    @functools.partial(
        pl.kernel,
        mesh=mesh,
        out_type=jax.ShapeDtypeStruct((nrows, C), jnp.float32),
        scratch_types=[
            pltpu.VMEM((tpw,), jnp.int32),
            pltpu.VMEM((tpw, C), jnp.float32),
            pltpu.SemaphoreType.DMA,
        ],
    )
    def _dispatch(x_hbm, sidx_hbm, disp_hbm, idx_v, rows_v, sem):
        wid = lax.axis_index("s") * info.num_cores + lax.axis_index("c")
        base = wid * tpw
        pltpu.sync_copy(sidx_hbm.at[pl.ds(base, tpw)], idx_v)
        pltpu.sync_copy(x_hbm.at[pl.ds(base, tpw)], rows_v)
        pltpu.async_copy(rows_v, disp_hbm.at[idx_v], sem).wait()

    disp = _dispatch(xf, sidx)

    nh = 4
    HB = H // nh
    eflat = pl.pallas_call(
        functools.partial(_ffn_body, cap=cap),
        grid=(E, nh),
        in_specs=[
            pl.BlockSpec((slots, C), lambda e, h: (e, 0)),
            pl.BlockSpec((1, C, HB), lambda e, h: (e, 0, h)),
            pl.BlockSpec((1, HB, C), lambda e, h: (e, h, 0)),
        ],
        out_specs=pl.BlockSpec((slots, C), lambda e, h: (e, 0)),
        out_shape=jax.ShapeDtypeStruct((E * slots, C), jnp.float32),
        compiler_params=pltpu.CompilerParams(
            dimension_semantics=("parallel", "arbitrary"),
        ),
    )(disp, W1, W2)

    @functools.partial(
        pl.kernel,
        mesh=mesh,
        out_type=jax.ShapeDtypeStruct((N, C), jnp.float32),
        scratch_types=[
            pltpu.VMEM((tpw,), jnp.int32),
            pltpu.VMEM((tpw, C), jnp.float32),
            pltpu.SemaphoreType.DMA,
        ],
    )
    def _combine(eo_hbm, gidx_hbm, y_hbm, idx_v, rows_v, sem):
        wid = lax.axis_index("s") * info.num_cores + lax.axis_index("c")
        base = wid * tpw
        pltpu.sync_copy(gidx_hbm.at[pl.ds(base, tpw)], idx_v)
        pltpu.async_copy(eo_hbm.at[idx_v], rows_v, sem).wait()
        pltpu.sync_copy(rows_v, y_hbm.at[pl.ds(base, tpw)])

    y = _combine(eflat, gidx)
    return y.reshape(Bv, Tv, C)


# speedup vs baseline: 1.1446x; 1.1446x over previous
import functools

import jax
import jax.numpy as jnp
from jax import lax
from jax.experimental import pallas as pl
from jax.experimental.pallas import tpu as pltpu
from jax.experimental.pallas import tpu_sc as plsc

_TOP_K = 1
_CAP_FACTOR = 1.25
_BLK = 256


def _router_body(noisy_ref, sidx_ref, gidx_ref, run_ref, *, cap, slots, E, trash):
    i = pl.program_id(0)

    @pl.when(i == 0)
    def _():
        run_ref[...] = jnp.zeros_like(run_ref)

    nb = noisy_ref[...]
    blk = nb.shape[0]
    m = jnp.max(nb, axis=1, keepdims=True)
    lane = lax.broadcasted_iota(jnp.int32, (blk, E), 1)
    eidx = jnp.min(jnp.where(nb == m, lane, E), axis=1, keepdims=True)
    onehot = (lane == eidx).astype(jnp.float32)
    r = lax.broadcasted_iota(jnp.int32, (blk, blk), 0)
    c = lax.broadcasted_iota(jnp.int32, (blk, blk), 1)
    ltri = (r >= c).astype(jnp.float32)
    incl = jnp.dot(ltri, onehot, preferred_element_type=jnp.float32)
    run = run_ref[0:1, :]
    pos = jnp.sum((incl + run - 1.0) * onehot, axis=1, keepdims=True).astype(jnp.int32)
    run_ref[0:1, :] = run + jnp.sum(onehot, axis=0, keepdims=True)
    base = eidx * slots
    sidx_ref[...] = jnp.where(pos < cap, base + pos, trash)
    gidx_ref[...] = base + jnp.minimum(pos, cap)


def _ffn_body(x_ref, w1_ref, w2_ref, out_ref, *, cap):
    h = pl.program_id(1)
    xb = x_ref[...]
    row = lax.broadcasted_iota(jnp.int32, (xb.shape[0], 1), 0)
    xb = jnp.where(row < cap, xb, 0.0)
    a = jnp.dot(
        xb.astype(jnp.bfloat16),
        w1_ref[0].astype(jnp.bfloat16),
        preferred_element_type=jnp.float32,
    )
    s = a * jax.nn.sigmoid(a)
    contrib = jnp.dot(
        s.astype(jnp.bfloat16),
        w2_ref[0].astype(jnp.bfloat16),
        preferred_element_type=jnp.float32,
    )

    @pl.when(h == 0)
    def _():
        out_ref[...] = jnp.zeros_like(out_ref)

    out_ref[...] += contrib


def kernel(x, Wr, Wn, W1, W2, noise):
    Bv, Tv, C = x.shape
    E = Wr.shape[1]
    H = W1.shape[2]
    N = Bv * Tv
    cap = int(N * _TOP_K / E * _CAP_FACTOR)
    slots = cap + 8
    trash = E * slots
    nrows = trash + 8

    logits = x @ Wr
    noise_logits = x @ Wn
    noisy = (logits + noise * jax.nn.softplus(noise_logits)).reshape(N, E)

    sidx, gidx = pl.pallas_call(
        functools.partial(_router_body, cap=cap, slots=slots, E=E, trash=trash),
        grid=(N // _BLK,),
        in_specs=[pl.BlockSpec((_BLK, E), lambda i: (i, 0))],
        out_specs=[
            pl.BlockSpec((_BLK, 1), lambda i: (i, 0)),
            pl.BlockSpec((_BLK, 1), lambda i: (i, 0)),
        ],
        out_shape=[jax.ShapeDtypeStruct((N, 1), jnp.int32)] * 2,
        scratch_shapes=[pltpu.VMEM((8, E), jnp.float32)],
    )(noisy)
    sidx = sidx.reshape(N)
    gidx = gidx.reshape(N)

    xf = x.reshape(N, C)
    info = plsc.get_sparse_core_info()
    NW = info.num_cores * info.num_subcores
    tpw = N // NW
    mesh = plsc.VectorSubcoreMesh(core_axis_name="c", subcore_axis_name="s")

    @functools.partial(
        pl.kernel,
        mesh=mesh,
        out_type=jax.ShapeDtypeStruct((nrows, C), jnp.float32),
        scratch_types=[
            pltpu.VMEM((tpw,), jnp.int32),
            pltpu.VMEM((tpw, C), jnp.float32),
            pltpu.SemaphoreType.DMA,
        ],
    )
    def _dispatch(x_hbm, sidx_hbm, disp_hbm, idx_v, rows_v, sem):
        wid = lax.axis_index("s") * info.num_cores + lax.axis_index("c")
        base = wid * tpw
        pltpu.sync_copy(sidx_hbm.at[pl.ds(base, tpw)], idx_v)
        pltpu.sync_copy(x_hbm.at[pl.ds(base, tpw)], rows_v)
        pltpu.async_copy(rows_v, disp_hbm.at[idx_v], sem).wait()

    disp = _dispatch(xf, sidx)

    nh = 2
    HB = H // nh
    eflat = pl.pallas_call(
        functools.partial(_ffn_body, cap=cap),
        grid=(E, nh),
        in_specs=[
            pl.BlockSpec((slots, C), lambda e, h: (e, 0)),
            pl.BlockSpec((1, C, HB), lambda e, h: (e, 0, h)),
            pl.BlockSpec((1, HB, C), lambda e, h: (e, h, 0)),
        ],
        out_specs=pl.BlockSpec((slots, C), lambda e, h: (e, 0)),
        out_shape=jax.ShapeDtypeStruct((E * slots, C), jnp.float32),
        compiler_params=pltpu.CompilerParams(
            dimension_semantics=("parallel", "arbitrary"),
        ),
    )(disp, W1, W2)

    @functools.partial(
        pl.kernel,
        mesh=mesh,
        out_type=jax.ShapeDtypeStruct((N, C), jnp.float32),
        scratch_types=[
            pltpu.VMEM((tpw,), jnp.int32),
            pltpu.VMEM((tpw, C), jnp.float32),
            pltpu.SemaphoreType.DMA,
        ],
    )
    def _combine(eo_hbm, gidx_hbm, y_hbm, idx_v, rows_v, sem):
        wid = lax.axis_index("s") * info.num_cores + lax.axis_index("c")
        base = wid * tpw
        pltpu.sync_copy(gidx_hbm.at[pl.ds(base, tpw)], idx_v)
        pltpu.async_copy(eo_hbm.at[idx_v], rows_v, sem).wait()
        pltpu.sync_copy(rows_v, y_hbm.at[pl.ds(base, tpw)])

    y = _combine(eflat, gidx)
    return y.reshape(Bv, Tv, C)


# speedup vs baseline: 1.1518x; 1.0063x over previous
import functools

import jax
import jax.numpy as jnp
from jax import lax
from jax.experimental import pallas as pl
from jax.experimental.pallas import tpu as pltpu
from jax.experimental.pallas import tpu_sc as plsc

_TOP_K = 1
_CAP_FACTOR = 1.25
_BLK = 512


def _router_body(noisy_ref, sidx_ref, gidx_ref, run_ref, *, cap, slots, E, trash):
    i = pl.program_id(0)

    @pl.when(i == 0)
    def _():
        run_ref[...] = jnp.zeros_like(run_ref)

    nb = noisy_ref[...]
    blk = nb.shape[0]
    m = jnp.max(nb, axis=1, keepdims=True)
    lane = lax.broadcasted_iota(jnp.int32, (blk, E), 1)
    eidx = jnp.min(jnp.where(nb == m, lane, E), axis=1, keepdims=True)
    onehot = (lane == eidx).astype(jnp.float32)
    r = lax.broadcasted_iota(jnp.int32, (blk, blk), 0)
    c = lax.broadcasted_iota(jnp.int32, (blk, blk), 1)
    ltri = (r >= c).astype(jnp.float32)
    incl = jnp.dot(ltri, onehot, preferred_element_type=jnp.float32)
    run = run_ref[0:1, :]
    pos = jnp.sum((incl + run - 1.0) * onehot, axis=1, keepdims=True).astype(jnp.int32)
    run_ref[0:1, :] = run + jnp.sum(onehot, axis=0, keepdims=True)
    base = eidx * slots
    sidx_ref[...] = jnp.where(pos < cap, base + pos, trash)
    gidx_ref[...] = base + jnp.minimum(pos, cap)


def _ffn_body(x_ref, w1_ref, w2_ref, out_ref, *, cap):
    h = pl.program_id(1)
    xb = x_ref[...]
    row = lax.broadcasted_iota(jnp.int32, (xb.shape[0], 1), 0)
    xb = jnp.where(row < cap, xb, 0.0)
    a = jnp.dot(xb, w1_ref[0], preferred_element_type=jnp.float32)
    s = a * jax.nn.sigmoid(a)
    contrib = jnp.dot(s, w2_ref[0], preferred_element_type=jnp.float32)

    @pl.when(h == 0)
    def _():
        out_ref[...] = jnp.zeros_like(out_ref)

    out_ref[...] += contrib


def kernel(x, Wr, Wn, W1, W2, noise):
    Bv, Tv, C = x.shape
    E = Wr.shape[1]
    H = W1.shape[2]
    N = Bv * Tv
    cap = int(N * _TOP_K / E * _CAP_FACTOR)
    slots = cap + 8
    trash = E * slots
    nrows = trash + 8

    logits = x @ Wr
    noise_logits = x @ Wn
    noisy = (logits + noise * jax.nn.softplus(noise_logits)).reshape(N, E)

    sidx, gidx = pl.pallas_call(
        functools.partial(_router_body, cap=cap, slots=slots, E=E, trash=trash),
        grid=(N // _BLK,),
        in_specs=[pl.BlockSpec((_BLK, E), lambda i: (i, 0))],
        out_specs=[
            pl.BlockSpec((_BLK, 1), lambda i: (i, 0)),
            pl.BlockSpec((_BLK, 1), lambda i: (i, 0)),
        ],
        out_shape=[jax.ShapeDtypeStruct((N, 1), jnp.int32)] * 2,
        scratch_shapes=[pltpu.VMEM((8, E), jnp.float32)],
    )(noisy)
    sidx = sidx.reshape(N)
    gidx = gidx.reshape(N)

    xf = x.reshape(N, C)
    info = plsc.get_sparse_core_info()
    NW = info.num_cores * info.num_subcores
    tpw = N // NW
    mesh = plsc.VectorSubcoreMesh(core_axis_name="c", subcore_axis_name="s")

    @functools.partial(
        pl.kernel,
        mesh=mesh,
        out_type=jax.ShapeDtypeStruct((nrows, C), jnp.float32),
        scratch_types=[
            pltpu.VMEM((tpw,), jnp.int32),
            pltpu.VMEM((tpw, C), jnp.float32),
            pltpu.SemaphoreType.DMA,
            pltpu.SemaphoreType.DMA,
        ],
    )
    def _dispatch(x_hbm, sidx_hbm, disp_hbm, idx_v, rows_v, sem, sem2):
        wid = lax.axis_index("s") * info.num_cores + lax.axis_index("c")
        base = wid * tpw
        cp_i = pltpu.async_copy(sidx_hbm.at[pl.ds(base, tpw)], idx_v, sem)
        cp_r = pltpu.async_copy(x_hbm.at[pl.ds(base, tpw)], rows_v, sem2)
        cp_i.wait()
        cp_r.wait()
        pltpu.async_copy(rows_v, disp_hbm.at[idx_v], sem).wait()

    disp = _dispatch(xf, sidx)

    nh = 2
    HB = H // nh
    eflat = pl.pallas_call(
        functools.partial(_ffn_body, cap=cap),
        grid=(E, nh),
        in_specs=[
            pl.BlockSpec((slots, C), lambda e, h: (e, 0)),
            pl.BlockSpec((1, C, HB), lambda e, h: (e, 0, h)),
            pl.BlockSpec((1, HB, C), lambda e, h: (e, h, 0)),
        ],
        out_specs=pl.BlockSpec((slots, C), lambda e, h: (e, 0)),
        out_shape=jax.ShapeDtypeStruct((E * slots, C), jnp.float32),
        compiler_params=pltpu.CompilerParams(
            dimension_semantics=("parallel", "arbitrary"),
        ),
    )(disp, W1, W2)

    @functools.partial(
        pl.kernel,
        mesh=mesh,
        out_type=jax.ShapeDtypeStruct((N, C), jnp.float32),
        scratch_types=[
            pltpu.VMEM((tpw,), jnp.int32),
            pltpu.VMEM((tpw, C), jnp.float32),
            pltpu.SemaphoreType.DMA,
        ],
    )
    def _combine(eo_hbm, gidx_hbm, y_hbm, idx_v, rows_v, sem):
        wid = lax.axis_index("s") * info.num_cores + lax.axis_index("c")
        base = wid * tpw
        pltpu.sync_copy(gidx_hbm.at[pl.ds(base, tpw)], idx_v)
        pltpu.async_copy(eo_hbm.at[idx_v], rows_v, sem).wait()
        pltpu.sync_copy(rows_v, y_hbm.at[pl.ds(base, tpw)])

    y = _combine(eflat, gidx)
    return y.reshape(Bv, Tv, C)
